# baseline (speedup 1.0000x reference)
import functools

import jax
import jax.numpy as jnp
from jax import lax
from jax.experimental import pallas as pl
from jax.experimental.pallas import tpu as pltpu
from jax.experimental.pallas import tpu_sc as plsc

D_IN = 4096
D_OUT = 4096
NUMEL = D_OUT * D_IN

NC = 2
NS = 16
NW = NC * NS
SB = 128
DEPTH = 8


def _copy_body(w_in, w_out):
    w_out[...] = w_in[...]


@functools.partial(jax.jit, static_argnums=())
def _tc_copy(w2d):
    blk = 256
    return pl.pallas_call(
        _copy_body,
        grid=(D_OUT // blk,),
        in_specs=[pl.BlockSpec((blk, D_IN), lambda i: (i, 0))],
        out_specs=pl.BlockSpec((blk, D_IN), lambda i: (i, 0)),
        out_shape=jax.ShapeDtypeStruct((D_OUT, D_IN), jnp.float32),
    )(w2d)


def _make_sc_scatter(K):
    mesh = plsc.VectorSubcoreMesh(
        core_axis_name="c", subcore_axis_name="s", num_cores=NC, num_subcores=NS
    )

    @functools.partial(
        pl.kernel,
        mesh=mesh,
        out_type=(),
        scratch_types=[
            pltpu.VMEM((K * SB,), jnp.int32),
            pltpu.VMEM((K * SB,), jnp.float32),
            pltpu.SemaphoreType.DMA,
        ],
    )
    def sc_scatter(w_hbm, idx_hbm, vals_hbm, idx_v, vals_v, sem):
        wid = lax.axis_index("s") * NC + lax.axis_index("c")
        pltpu.sync_copy(idx_hbm.at[wid], idx_v)
        pltpu.sync_copy(vals_hbm.at[wid], vals_v)
        pltpu.async_copy(vals_v, w_hbm.at[idx_v], sem).wait()

    return sc_scatter


def _mm_body(x_ref, w_ref, b_ref, o_ref):
    acc = lax.dot_general(
        x_ref[...],
        w_ref[...],
        dimension_numbers=(((1,), (1,)), ((), ())),
        preferred_element_type=jnp.float32,
    )
    o_ref[...] = acc + b_ref[...][None, :]


def _tc_matmul(x, w2d, b):
    bn = 512
    batch = x.shape[0]
    return pl.pallas_call(
        _mm_body,
        grid=(D_OUT // bn,),
        in_specs=[
            pl.BlockSpec((batch, D_IN), lambda i: (0, 0)),
            pl.BlockSpec((bn, D_IN), lambda i: (i, 0)),
            pl.BlockSpec((bn,), lambda i: (i,)),
        ],
        out_specs=pl.BlockSpec((batch, bn), lambda i: (0, i)),
        out_shape=jax.ShapeDtypeStruct((batch, D_OUT), jnp.float32),
    )(x, w2d, b)


def kernel(x, W_flat, b, flip_vals, flip_idx):
    n = flip_idx.shape[0]
    chunk = NW * SB
    K = -(-n // chunk)
    npad = K * chunk - n

    idx = flip_idx.astype(jnp.int32)
    vals = flip_vals.astype(jnp.float32)
    if npad:
        idx = jnp.concatenate([idx, jnp.broadcast_to(idx[0], (npad,))])
        vals = jnp.concatenate([vals, jnp.broadcast_to(vals[0], (npad,))])
    idx3 = idx.reshape(NW, K * SB)
    vals3 = vals.reshape(NW, K * SB)

    w_mod = _tc_copy(W_flat.reshape(D_OUT, D_IN))
    wref = jax.new_ref(w_mod.reshape(NUMEL))
    _make_sc_scatter(K)(wref, idx3, vals3)
    w_final = jax.freeze(wref)

    return _tc_matmul(x, w_final.reshape(D_OUT, D_IN), b)


# speedup vs baseline: 2.1751x; 2.1751x over previous
import functools

import jax
import jax.numpy as jnp
from jax import lax
from jax.experimental import pallas as pl
from jax.experimental.pallas import tpu as pltpu
from jax.experimental.pallas import tpu_sc as plsc

D_IN = 4096
D_OUT = 4096
NUMEL = D_OUT * D_IN

NC = 2
NS = 16
NW = NC * NS
LANES = 16

REG_W = 1 << 20
NREG = NUMEL // REG_W
NPASS = NREG // NC
SLICE = REG_W // NS
DUMP = 2048
SPM_W = REG_W + 2 * DUMP
HOP_W = 1 << 14
HOPS = SLICE // HOP_W
WIN = 5904


def _make_sc_fused(NWIN):
    NV = WIN // LANES
    mesh = plsc.VectorSubcoreMesh(
        core_axis_name="c", subcore_axis_name="s", num_cores=NC, num_subcores=NS
    )

    @functools.partial(
        pl.kernel,
        mesh=mesh,
        out_type=jax.ShapeDtypeStruct((NUMEL,), jnp.float32),
        scratch_types=[
            pltpu.VMEM((WIN,), jnp.int32),
            pltpu.VMEM((WIN,), jnp.float32),
            pltpu.VMEM((WIN,), jnp.int32),
            pltpu.VMEM((HOP_W,), jnp.float32),
            pltpu.VMEM_SHARED((SPM_W,), jnp.float32),
        ],
    )
    def sc_fused(w_hbm, idx_hbm, vals_hbm, wmod_hbm, idxw, valsw, ibuf,
                 hop, spmem):
        cid = lax.axis_index("c")
        sid = lax.axis_index("s")

        def pass_body(p, c0):
            rbase = (p * NC + cid) * REG_W
            def hop_in(h, c3):
                o = sid * SLICE + h * HOP_W
                pltpu.sync_copy(w_hbm.at[pl.ds(rbase + o, HOP_W)], hop)
                pltpu.sync_copy(hop, spmem.at[pl.ds(DUMP + o, HOP_W)])
                return c3

            lax.fori_loop(0, HOPS, hop_in, 0)
            plsc.subcore_barrier()

            def win_body(w, c1):
                pltpu.sync_copy(idx_hbm.at[sid, w], idxw)
                pltpu.sync_copy(vals_hbm.at[sid, w], valsw)

                def vbody(i, c2):
                    iv = idxw[pl.ds(i * LANES, LANES)]
                    loc = iv - rbase
                    spread = iv & jnp.int32(DUMP - 1)
                    loc = lax.max(loc, spread - jnp.int32(DUMP))
                    loc = lax.min(loc, spread + jnp.int32(REG_W))
                    ibuf[pl.ds(i * LANES, LANES)] = loc + jnp.int32(DUMP)
                    return c2

                lax.fori_loop(0, NV, vbody, 0)
                pltpu.sync_copy(valsw, spmem.at[ibuf], add=True)
                return c1

            lax.fori_loop(0, NWIN, win_body, 0)
            plsc.subcore_barrier()

            def hop_out(h, c4):
                o = sid * SLICE + h * HOP_W
                pltpu.sync_copy(spmem.at[pl.ds(DUMP + o, HOP_W)], hop)
                pltpu.sync_copy(hop, wmod_hbm.at[pl.ds(rbase + o, HOP_W)])
                return c4

            lax.fori_loop(0, HOPS, hop_out, 0)
            plsc.subcore_barrier()
            return c0

        lax.fori_loop(0, NPASS, pass_body, 0)

    return sc_fused


def _mm_body(x_ref, w_ref, b_ref, o_ref):
    acc = lax.dot_general(
        x_ref[...],
        w_ref[...],
        dimension_numbers=(((1,), (1,)), ((), ())),
        preferred_element_type=jnp.float32,
    )
    o_ref[...] = acc + b_ref[...][None, :]


def _tc_matmul(x, w2d, b):
    bn = 512
    batch = x.shape[0]
    return pl.pallas_call(
        _mm_body,
        grid=(D_OUT // bn,),
        in_specs=[
            pl.BlockSpec((batch, D_IN), lambda i: (0, 0)),
            pl.BlockSpec((bn, D_IN), lambda i: (i, 0)),
            pl.BlockSpec((bn,), lambda i: (i,)),
        ],
        out_specs=pl.BlockSpec((batch, bn), lambda i: (0, i)),
        out_shape=jax.ShapeDtypeStruct((batch, D_OUT), jnp.float32),
    )(x, w2d, b)


def kernel(x, W_flat, b, flip_vals, flip_idx):
    n = flip_idx.shape[0]
    chunk_q = NS * WIN
    nwin = -(-n // chunk_q)
    CH = nwin * WIN
    npad = CH * NS - n

    idx = flip_idx.astype(jnp.int32)
    vals = flip_vals.astype(jnp.float32)
    if npad:
        idx = jnp.concatenate([idx, jnp.broadcast_to(idx[0], (npad,))])
        vals = jnp.concatenate([vals, jnp.zeros((npad,), jnp.float32)])
    idx3 = idx.reshape(NS, nwin, WIN)
    vals3 = vals.reshape(NS, nwin, WIN)

    w_mod = _make_sc_fused(nwin)(W_flat, idx3, vals3)
    return _tc_matmul(x, w_mod.reshape(D_OUT, D_IN), b)
